# baseline (speedup 1.0000x reference)
import functools

import jax
import jax.numpy as jnp
from jax.experimental import pallas as pl
from jax.experimental.pallas import tpu as pltpu
from jax.experimental.pallas import tpu_sc as plsc

HALF = 200000
DIM = 64
DM = 1024
WINDOW = 128
REG = 16

C_ROWS = 8000


def _tc_compact(table):
    def body(a_ref, b_ref, o_ref):
        o_ref[:, :DIM] = a_ref[...]
        o_ref[:, DIM:] = b_ref[...]

    return pl.pallas_call(
        body,
        grid=(HALF // C_ROWS,),
        in_specs=[
            pl.BlockSpec((C_ROWS, DIM), lambda i: (i, 0)),
            pl.BlockSpec((C_ROWS, DIM), lambda i: (i + HALF // C_ROWS, 0)),
        ],
        out_specs=pl.BlockSpec((C_ROWS, 2 * DIM), lambda i: (i, 0)),
        out_shape=jax.ShapeDtypeStruct((HALF, 2 * DIM), jnp.float32),
        compiler_params=pltpu.CompilerParams(
            dimension_semantics=("parallel",),
        ),
    )(table, table)


def _sc_hash_gather(x2, p2, t2):
    n = x2.shape[1]
    mesh = plsc.VectorSubcoreMesh(core_axis_name="c", subcore_axis_name="s")

    @functools.partial(
        pl.kernel,
        out_type=jax.ShapeDtypeStruct((n, 2 * DIM), jnp.float32),
        mesh=mesh,
        scratch_types=[pltpu.VMEM((1, WINDOW), jnp.int32)],
    )
    def k(x_hbm, p_hbm, t_hbm, o_hbm, idx_s):
        def body(x_v, p_v, o_v):
            @pl.loop(0, WINDOW, step=REG)
            def _(c):
                slc = (pl.ds(0, 1), pl.ds(c, REG))
                h = x_v.at[*slc][...] * 3 + p_v.at[*slc][...]
                idx_s.at[*slc][...] = jnp.where(h >= HALF, h - HALF, h)

            pltpu.sync_copy(t_hbm.at[idx_s.at[0]], o_v)

        pltpu.emit_pipeline(
            body,
            grid=(n // WINDOW,),
            in_specs=[
                pl.BlockSpec((1, WINDOW), lambda i: (0, i)),
                pl.BlockSpec((1, WINDOW), lambda i: (0, i)),
            ],
            out_specs=[pl.BlockSpec((WINDOW, 2 * DIM), lambda i: (i, 0))],
            core_axis_name=("c", "s"),
            dimension_semantics=(pltpu.PARALLEL,),
        )(x_hbm, p_hbm, o_hbm)

    return k(x2, p2, t2)


def _tc_project(emb, xc, pc, w):
    n = emb.shape[0]
    rows = 2048

    def body(e_ref, x_ref, p_ref, w_ref, o_ref):
        h = x_ref[...] * 3 + p_ref[...]
        e = jnp.where(h >= HALF, e_ref[:, DIM:], e_ref[:, :DIM])
        o_ref[...] = jax.lax.dot_general(
            e, w_ref[...],
            (((1,), (1,)), ((), ())),
            preferred_element_type=jnp.float32,
        )

    return pl.pallas_call(
        body,
        grid=(n // rows,),
        in_specs=[
            pl.BlockSpec((rows, 2 * DIM), lambda i: (i, 0)),
            pl.BlockSpec((rows, 1), lambda i: (i, 0)),
            pl.BlockSpec((rows, 1), lambda i: (i, 0)),
            pl.BlockSpec((DM, DIM), lambda i: (0, 0)),
        ],
        out_specs=pl.BlockSpec((rows, DM), lambda i: (i, 0)),
        out_shape=jax.ShapeDtypeStruct((n, DM), jnp.float32),
        compiler_params=pltpu.CompilerParams(
            dimension_semantics=("parallel",),
        ),
    )(emb, xc, pc, w)


def kernel(x, table, W):
    b, s = x.shape
    x32 = x.astype(jnp.int32)
    prev = jnp.roll(x32, 1, axis=1).at[:, 0].set(0)
    n = b * s
    with jax.enable_x64(False):
        t2 = _tc_compact(table)
        emb = _sc_hash_gather(x32.reshape(1, n), prev.reshape(1, n), t2)
        out = _tc_project(
            emb, x32.reshape(n, 1), prev.reshape(n, 1), W
        )
    return out.reshape(b, s, DM)


# speedup vs baseline: 1.3492x; 1.3492x over previous
import functools

import jax
import jax.numpy as jnp
from jax.experimental import pallas as pl
from jax.experimental.pallas import tpu as pltpu
from jax.experimental.pallas import tpu_sc as plsc

HALF = 200000
DIM = 64
DM = 1024
WINDOW = 128
REG = 16

C_CHUNKS = 8


def _tc_compact(table):
    rows = HALF // C_CHUNKS

    def body(tab_ref, o_ref, sem):
        copies = []
        for c in range(C_CHUNKS):
            lo = pltpu.make_async_copy(
                tab_ref.at[pl.ds(c * rows, rows), :],
                o_ref.at[pl.ds(c * rows, rows), pl.ds(0, DIM)],
                sem,
            )
            hi = pltpu.make_async_copy(
                tab_ref.at[pl.ds(HALF + c * rows, rows), :],
                o_ref.at[pl.ds(c * rows, rows), pl.ds(DIM, DIM)],
                sem,
            )
            lo.start()
            hi.start()
            copies += [lo, hi]
        for cp in copies:
            cp.wait()

    return pl.pallas_call(
        body,
        in_specs=[pl.BlockSpec(memory_space=pl.ANY)],
        out_specs=pl.BlockSpec(memory_space=pl.ANY),
        out_shape=jax.ShapeDtypeStruct((HALF, 2 * DIM), jnp.float32),
        scratch_shapes=[pltpu.SemaphoreType.DMA],
    )(table)


def _sc_hash_gather(x2, p2, t2):
    n = x2.shape[1]
    mesh = plsc.VectorSubcoreMesh(core_axis_name="c", subcore_axis_name="s")

    @functools.partial(
        pl.kernel,
        out_type=jax.ShapeDtypeStruct((n, 2 * DIM), jnp.float32),
        mesh=mesh,
        scratch_types=[pltpu.VMEM((1, WINDOW), jnp.int32)],
    )
    def k(x_hbm, p_hbm, t_hbm, o_hbm, idx_s):
        def body(x_v, p_v, o_v):
            @pl.loop(0, WINDOW, step=REG)
            def _(c):
                slc = (pl.ds(0, 1), pl.ds(c, REG))
                h = x_v.at[*slc][...] * 3 + p_v.at[*slc][...]
                idx_s.at[*slc][...] = jax.lax.shift_right_logical(h, 1)

            pltpu.sync_copy(t_hbm.at[idx_s.at[0]], o_v)

        pltpu.emit_pipeline(
            body,
            grid=(n // WINDOW,),
            in_specs=[
                pl.BlockSpec((1, WINDOW), lambda i: (0, i)),
                pl.BlockSpec((1, WINDOW), lambda i: (0, i)),
            ],
            out_specs=[pl.BlockSpec((WINDOW, 2 * DIM), lambda i: (i, 0))],
            core_axis_name=("c", "s"),
            dimension_semantics=(pltpu.PARALLEL,),
        )(x_hbm, p_hbm, o_hbm)

    return k(x2, p2, t2)


def _tc_project(emb, xc, pc, w):
    n = emb.shape[0]
    rows = 2048

    def body(e_ref, x_ref, p_ref, w_ref, o_ref):
        h = x_ref[...] * 3 + p_ref[...]
        e = jnp.where((h & 1) == 1, e_ref[:, DIM:], e_ref[:, :DIM])
        o_ref[...] = jax.lax.dot_general(
            e, w_ref[...],
            (((1,), (1,)), ((), ())),
            preferred_element_type=jnp.float32,
        )

    return pl.pallas_call(
        body,
        grid=(n // rows,),
        in_specs=[
            pl.BlockSpec((rows, 2 * DIM), lambda i: (i, 0)),
            pl.BlockSpec((rows, 1), lambda i: (i, 0)),
            pl.BlockSpec((rows, 1), lambda i: (i, 0)),
            pl.BlockSpec((DM, DIM), lambda i: (0, 0)),
        ],
        out_specs=pl.BlockSpec((rows, DM), lambda i: (i, 0)),
        out_shape=jax.ShapeDtypeStruct((n, DM), jnp.float32),
        compiler_params=pltpu.CompilerParams(
            dimension_semantics=("parallel",),
        ),
    )(emb, xc, pc, w)


def kernel(x, table, W):
    b, s = x.shape
    x32 = x.astype(jnp.int32)
    prev = jnp.roll(x32, 1, axis=1).at[:, 0].set(0)
    n = b * s
    with jax.enable_x64(False):
        t2 = table.reshape(500000, 2 * DIM)[:HALF]
        emb = _sc_hash_gather(x32.reshape(1, n), prev.reshape(1, n), t2)
        out = _tc_project(
            emb, x32.reshape(n, 1), prev.reshape(n, 1), W
        )
    return out.reshape(b, s, DM)
